# Initial kernel scaffold; baseline (speedup 1.0000x reference)
import jax
import jax.numpy as jnp
from jax.experimental import pallas as pl


def kernel(x, edge_index, W1, b1, Wl0, bl0, Wr0, br0, Wl1, bl1, Wr1, br1, g0, be0, g1, be1, g2, be2, W2, b2):
    raise NotImplementedError("write your pallas kernel here")


# speedup vs baseline: 5.4692x; 5.4692x over previous
import functools

import jax
import jax.numpy as jnp
from jax import lax
from jax.experimental import pallas as pl
from jax.experimental.pallas import tpu as pltpu
from jax.experimental.pallas import tpu_sc as plsc

N = 10000
E = 320000
D = 128
H = 128
OUT = 128
BN_EPS = 1e-5

NC = 2
NS = 16
NPAD = 10240
ROWS_PER_TILE = NPAD // NS
CHUNK = 80
EDGES_PER_CORE = E // NC
EDGES_PER_TILE = E // (NC * NS)
NCHUNK = EDGES_PER_TILE // CHUNK

_mesh = plsc.VectorSubcoreMesh(core_axis_name="c", subcore_axis_name="s")


def _sc_agg_body(with_deg, *refs):
    if with_deg:
        (h_hbm, src_hbm, dst_hbm, z128_hbm, z1_hbm,
         agg_out, deg_out,
         agg_sh, deg_sh, src_v, dst_v, rows_v, ones_v, sem) = refs
    else:
        (h_hbm, src_hbm, dst_hbm, z128_hbm,
         agg_out,
         agg_sh, src_v, dst_v, rows_v, sem) = refs

    cid = lax.axis_index("c")
    sid = lax.axis_index("s")
    r0 = sid * ROWS_PER_TILE

    pltpu.sync_copy(z128_hbm.at[pl.ds(r0, ROWS_PER_TILE)],
                    agg_sh.at[pl.ds(r0, ROWS_PER_TILE)])
    if with_deg:
        pltpu.sync_copy(z1_hbm.at[pl.ds(r0, ROWS_PER_TILE)],
                        deg_sh.at[pl.ds(r0, ROWS_PER_TILE)])

        @pl.loop(0, CHUNK // 16)
        def _(r):
            ones_v[pl.ds(r * 16, 16)] = jnp.ones((16,), jnp.float32)

    plsc.subcore_barrier()

    base = cid * EDGES_PER_CORE + sid * EDGES_PER_TILE

    @pl.loop(0, NCHUNK)
    def _(k):
        off = base + k * CHUNK
        pltpu.sync_copy(src_hbm.at[pl.ds(off, CHUNK)], src_v)
        pltpu.sync_copy(dst_hbm.at[pl.ds(off, CHUNK)], dst_v)
        pltpu.async_copy(h_hbm.at[src_v], rows_v, sem).wait()
        pltpu.sync_copy(rows_v, agg_sh.at[dst_v], add=True)
        if with_deg:
            pltpu.sync_copy(ones_v, deg_sh.at[dst_v], add=True)

    plsc.subcore_barrier()

    pltpu.sync_copy(agg_sh.at[pl.ds(r0, ROWS_PER_TILE)],
                    agg_out.at[cid, pl.ds(r0, ROWS_PER_TILE)])
    if with_deg:
        pltpu.sync_copy(deg_sh.at[pl.ds(r0, ROWS_PER_TILE)],
                        deg_out.at[cid, pl.ds(r0, ROWS_PER_TILE)])


def _sc_agg(h, src, dst, z128, z1, with_deg):
    out_type = [jax.ShapeDtypeStruct((NC, NPAD, D), jnp.float32)]
    scratch = [
        pltpu.VMEM_SHARED((NPAD, D), jnp.float32),
    ]
    if with_deg:
        out_type.append(jax.ShapeDtypeStruct((NC, NPAD), jnp.float32))
        scratch.append(pltpu.VMEM_SHARED((NPAD,), jnp.float32))
    scratch += [
        pltpu.VMEM((CHUNK,), jnp.int32),
        pltpu.VMEM((CHUNK,), jnp.int32),
        pltpu.VMEM((CHUNK, D), jnp.float32),
    ]
    if with_deg:
        scratch.append(pltpu.VMEM((CHUNK,), jnp.float32))
    scratch.append(pltpu.SemaphoreType.DMA)

    kern = pl.kernel(
        functools.partial(_sc_agg_body, with_deg),
        out_type=tuple(out_type) if len(out_type) > 1 else out_type[0],
        mesh=_mesh,
        scratch_types=scratch,
    )
    if with_deg:
        return kern(h, src, dst, z128, z1)
    return kern(h, src, dst, z128)


def _bn_scale(g):
    return g * (1.0 / jnp.sqrt(1.0 + BN_EPS))


BLK = 2000


def _tc1_body(x_ref, w1_ref, b1_ref, g0_ref, be0_ref, wr0_ref, br0_ref,
              h_ref, hr_ref):
    h = jnp.dot(x_ref[...], w1_ref[...], preferred_element_type=jnp.float32)
    h = (h + b1_ref[...]) * _bn_scale(g0_ref[...]) + be0_ref[...]
    h_ref[...] = h
    hr_ref[...] = jnp.dot(h, wr0_ref[...],
                          preferred_element_type=jnp.float32) + br0_ref[...]


def _tc_mid_body(agg_ref, deg_ref, hr_ref, wl_ref, bl_ref, g_ref, be_ref,
                 wr_ref, br_ref, h_ref, hrn_ref):
    mean = (agg_ref[0] + agg_ref[1]) / jnp.maximum(deg_ref[...], 1.0)
    t = jnp.dot(mean, wl_ref[...], preferred_element_type=jnp.float32)
    t = t + bl_ref[...] + hr_ref[...]
    h = t * _bn_scale(g_ref[...]) + be_ref[...]
    h_ref[...] = h
    hrn_ref[...] = jnp.dot(h, wr_ref[...],
                           preferred_element_type=jnp.float32) + br_ref[...]


def _tc_last_body(agg_ref, deg_ref, hr_ref, wl_ref, bl_ref, g_ref, be_ref,
                  w2_ref, b2_ref, out_ref, emb_ref):
    mean = (agg_ref[0] + agg_ref[1]) / jnp.maximum(deg_ref[...], 1.0)
    t = jnp.dot(mean, wl_ref[...], preferred_element_type=jnp.float32)
    t = t + bl_ref[...] + hr_ref[...]
    h = t * _bn_scale(g_ref[...]) + be_ref[...]
    emb = jnp.dot(h, w2_ref[...], preferred_element_type=jnp.float32)
    emb = emb + b2_ref[...]
    emb_ref[...] = emb
    m = jnp.max(emb, axis=1, keepdims=True)
    lse = jnp.log(jnp.sum(jnp.exp(emb - m), axis=1, keepdims=True))
    out_ref[...] = emb - m - lse


def _row_spec():
    return pl.BlockSpec((BLK, 128), lambda i: (i, 0))


def _w_spec():
    return pl.BlockSpec((128, 128), lambda i: (0, 0))


def _v_spec():
    return pl.BlockSpec((1, 128), lambda i: (0, 0))


def _agg_spec():
    return pl.BlockSpec((NC, BLK, 128), lambda i: (0, i, 0))


def _deg_spec():
    return _row_spec()


def _row_out(dtype=jnp.float32):
    return jax.ShapeDtypeStruct((N, 128), dtype)


def kernel(x, edge_index, W1, b1, Wl0, bl0, Wr0, br0, Wl1, bl1, Wr1, br1,
           g0, be0, g1, be1, g2, be2, W2, b2):
    src = edge_index[0]
    dst = edge_index[1]
    z128 = jnp.zeros((NPAD, D), jnp.float32)
    z1 = jnp.zeros((NPAD,), jnp.float32)

    row = lambda v: v.reshape(1, 128)
    grid = N // BLK

    h, hr = pl.pallas_call(
        _tc1_body,
        grid=(grid,),
        in_specs=[_row_spec(), _w_spec(), _v_spec(), _v_spec(), _v_spec(),
                  _w_spec(), _v_spec()],
        out_specs=[_row_spec(), _row_spec()],
        out_shape=[_row_out(), _row_out()],
    )(x, W1, row(b1), row(g0), row(be0), Wr0, row(br0))

    agg0, deg = _sc_agg(h, src, dst, z128, z1, with_deg=True)
    degb = jnp.broadcast_to((deg[0, :N] + deg[1, :N])[:, None], (N, 128))

    h1, hr1 = pl.pallas_call(
        _tc_mid_body,
        grid=(grid,),
        in_specs=[_agg_spec(), _deg_spec(), _row_spec(), _w_spec(), _v_spec(),
                  _v_spec(), _v_spec(), _w_spec(), _v_spec()],
        out_specs=[_row_spec(), _row_spec()],
        out_shape=[_row_out(), _row_out()],
    )(agg0, degb, hr, Wl0, row(bl0), row(g1), row(be1), Wr1, row(br1))

    agg1 = _sc_agg(h1, src, dst, z128, None, with_deg=False)

    out, emb = pl.pallas_call(
        _tc_last_body,
        grid=(grid,),
        in_specs=[_agg_spec(), _deg_spec(), _row_spec(), _w_spec(), _v_spec(),
                  _v_spec(), _v_spec(), _w_spec(), _v_spec()],
        out_specs=[_row_spec(), _row_spec()],
        out_shape=[_row_out(), _row_out()],
    )(agg1, degb, hr1, Wl1, row(bl1), row(g2), row(be2), W2, row(b2))

    return (out, emb)


# speedup vs baseline: 12.0745x; 2.2077x over previous
import functools

import jax
import jax.numpy as jnp
from jax import lax
from jax.experimental import pallas as pl
from jax.experimental.pallas import tpu as pltpu
from jax.experimental.pallas import tpu_sc as plsc

N = 10000
E = 320000
D = 128
H = 128
OUT = 128
BN_EPS = 1e-5

NC = 2
NS = 16
NPAD = 10240
ROWS_PER_TILE = NPAD // NS
CHUNK = 80
EDGES_PER_CORE = E // NC
EDGES_PER_TILE = E // (NC * NS)
NCHUNK = EDGES_PER_TILE // CHUNK

_mesh = plsc.VectorSubcoreMesh(core_axis_name="c", subcore_axis_name="s")


def _sc_agg_body(with_deg, *refs):
    if with_deg:
        (h_hbm, src_hbm, dst_hbm, z128_hbm, z1_hbm,
         agg_out, deg_out,
         agg_sh, deg_sh, src_big, dst_big, dstb0, dstb1,
         rows0, rows1, ones_v, sem0, sem1) = refs
    else:
        (h_hbm, src_hbm, dst_hbm, z128_hbm,
         agg_out,
         agg_sh, src_big, dst_big, dstb0, dstb1,
         rows0, rows1, sem0, sem1) = refs

    cid = lax.axis_index("c")
    sid = lax.axis_index("s")
    r0 = sid * ROWS_PER_TILE

    pltpu.sync_copy(z128_hbm.at[pl.ds(r0, ROWS_PER_TILE)],
                    agg_sh.at[pl.ds(r0, ROWS_PER_TILE)])
    if with_deg:
        pltpu.sync_copy(z1_hbm.at[pl.ds(r0, ROWS_PER_TILE)],
                        deg_sh.at[pl.ds(r0, ROWS_PER_TILE)])

        @pl.loop(0, CHUNK // 16)
        def _(r):
            ones_v[pl.ds(r * 16, 16)] = jnp.ones((16,), jnp.float32)

    base = cid * EDGES_PER_CORE + sid * EDGES_PER_TILE
    pltpu.sync_copy(src_hbm.at[pl.ds(base, EDGES_PER_TILE)], src_big)
    pltpu.sync_copy(dst_hbm.at[pl.ds(base, EDGES_PER_TILE)], dst_big)

    plsc.subcore_barrier()

    def prep_dst(dstb, c):
        @pl.loop(0, CHUNK // 16)
        def _(i):
            dstb[pl.ds(i * 16, 16)] = dst_big[pl.ds(c * CHUNK + i * 16, 16)]

    def gather(c, rows, sem):
        return pltpu.make_async_copy(
            h_hbm.at[src_big.at[pl.ds(c * CHUNK, CHUNK)]], rows, sem)

    def scatter(rows, dstb):
        pltpu.sync_copy(rows, agg_sh.at[dstb], add=True)
        if with_deg:
            pltpu.sync_copy(ones_v, deg_sh.at[dstb], add=True)

    prep_dst(dstb0, 0)
    gather(0, rows0, sem0).start()

    @pl.loop(0, (NCHUNK - 1) // 2)
    def _(k):
        c0 = 2 * k
        prep_dst(dstb1, c0 + 1)
        gather(c0 + 1, rows1, sem1).start()
        gather(c0, rows0, sem0).wait()
        scatter(rows0, dstb0)
        prep_dst(dstb0, c0 + 2)
        gather(c0 + 2, rows0, sem0).start()
        gather(c0 + 1, rows1, sem1).wait()
        scatter(rows1, dstb1)

    gather(NCHUNK - 1, rows0, sem0).wait()
    scatter(rows0, dstb0)

    plsc.subcore_barrier()

    pltpu.sync_copy(agg_sh.at[pl.ds(r0, ROWS_PER_TILE)],
                    agg_out.at[cid, pl.ds(r0, ROWS_PER_TILE)])
    if with_deg:
        pltpu.sync_copy(deg_sh.at[pl.ds(r0, ROWS_PER_TILE)],
                        deg_out.at[cid, pl.ds(r0, ROWS_PER_TILE)])


def _sc_agg(h, src, dst, z128, z1, with_deg):
    out_type = [jax.ShapeDtypeStruct((NC, NPAD, D), jnp.float32)]
    scratch = [
        pltpu.VMEM_SHARED((NPAD, D), jnp.float32),
    ]
    if with_deg:
        out_type.append(jax.ShapeDtypeStruct((NC, NPAD), jnp.float32))
        scratch.append(pltpu.VMEM_SHARED((NPAD,), jnp.float32))
    scratch += [
        pltpu.VMEM((EDGES_PER_TILE,), jnp.int32),
        pltpu.VMEM((EDGES_PER_TILE,), jnp.int32),
        pltpu.VMEM((CHUNK,), jnp.int32),
        pltpu.VMEM((CHUNK,), jnp.int32),
        pltpu.VMEM((CHUNK, D), jnp.float32),
        pltpu.VMEM((CHUNK, D), jnp.float32),
    ]
    if with_deg:
        scratch.append(pltpu.VMEM((CHUNK,), jnp.float32))
    scratch += [pltpu.SemaphoreType.DMA, pltpu.SemaphoreType.DMA]

    kern = pl.kernel(
        functools.partial(_sc_agg_body, with_deg),
        out_type=tuple(out_type) if len(out_type) > 1 else out_type[0],
        mesh=_mesh,
        scratch_types=scratch,
    )
    if with_deg:
        return kern(h, src, dst, z128, z1)
    return kern(h, src, dst, z128)


def _bn_scale(g):
    return g * (1.0 / jnp.sqrt(1.0 + BN_EPS))


BLK = 2000


def _tc1_body(x_ref, w1_ref, b1_ref, g0_ref, be0_ref, wr0_ref, br0_ref,
              h_ref, hr_ref):
    h = jnp.dot(x_ref[...], w1_ref[...], preferred_element_type=jnp.float32)
    h = (h + b1_ref[...]) * _bn_scale(g0_ref[...]) + be0_ref[...]
    h_ref[...] = h
    hr_ref[...] = jnp.dot(h, wr0_ref[...],
                          preferred_element_type=jnp.float32) + br0_ref[...]


def _tc_mid_body(agg_ref, deg_ref, hr_ref, wl_ref, bl_ref, g_ref, be_ref,
                 wr_ref, br_ref, h_ref, hrn_ref):
    mean = (agg_ref[0] + agg_ref[1]) / jnp.maximum(deg_ref[...], 1.0)
    t = jnp.dot(mean, wl_ref[...], preferred_element_type=jnp.float32)
    t = t + bl_ref[...] + hr_ref[...]
    h = t * _bn_scale(g_ref[...]) + be_ref[...]
    h_ref[...] = h
    hrn_ref[...] = jnp.dot(h, wr_ref[...],
                           preferred_element_type=jnp.float32) + br_ref[...]


def _tc_last_body(agg_ref, deg_ref, hr_ref, wl_ref, bl_ref, g_ref, be_ref,
                  w2_ref, b2_ref, out_ref, emb_ref):
    mean = (agg_ref[0] + agg_ref[1]) / jnp.maximum(deg_ref[...], 1.0)
    t = jnp.dot(mean, wl_ref[...], preferred_element_type=jnp.float32)
    t = t + bl_ref[...] + hr_ref[...]
    h = t * _bn_scale(g_ref[...]) + be_ref[...]
    emb = jnp.dot(h, w2_ref[...], preferred_element_type=jnp.float32)
    emb = emb + b2_ref[...]
    emb_ref[...] = emb
    m = jnp.max(emb, axis=1, keepdims=True)
    lse = jnp.log(jnp.sum(jnp.exp(emb - m), axis=1, keepdims=True))
    out_ref[...] = emb - m - lse


def _row_spec():
    return pl.BlockSpec((BLK, 128), lambda i: (i, 0))


def _w_spec():
    return pl.BlockSpec((128, 128), lambda i: (0, 0))


def _v_spec():
    return pl.BlockSpec((1, 128), lambda i: (0, 0))


def _agg_spec():
    return pl.BlockSpec((NC, BLK, 128), lambda i: (0, i, 0))


def _deg_spec():
    return _row_spec()


def _row_out(dtype=jnp.float32):
    return jax.ShapeDtypeStruct((N, 128), dtype)


def kernel(x, edge_index, W1, b1, Wl0, bl0, Wr0, br0, Wl1, bl1, Wr1, br1,
           g0, be0, g1, be1, g2, be2, W2, b2):
    src = edge_index[0]
    dst = edge_index[1]
    z128 = jnp.zeros((NPAD, D), jnp.float32)
    z1 = jnp.zeros((NPAD,), jnp.float32)

    row = lambda v: v.reshape(1, 128)
    grid = N // BLK

    h, hr = pl.pallas_call(
        _tc1_body,
        grid=(grid,),
        in_specs=[_row_spec(), _w_spec(), _v_spec(), _v_spec(), _v_spec(),
                  _w_spec(), _v_spec()],
        out_specs=[_row_spec(), _row_spec()],
        out_shape=[_row_out(), _row_out()],
    )(x, W1, row(b1), row(g0), row(be0), Wr0, row(br0))

    agg0, deg = _sc_agg(h, src, dst, z128, z1, with_deg=True)
    degb = jnp.broadcast_to((deg[0, :N] + deg[1, :N])[:, None], (N, 128))

    h1, hr1 = pl.pallas_call(
        _tc_mid_body,
        grid=(grid,),
        in_specs=[_agg_spec(), _deg_spec(), _row_spec(), _w_spec(), _v_spec(),
                  _v_spec(), _v_spec(), _w_spec(), _v_spec()],
        out_specs=[_row_spec(), _row_spec()],
        out_shape=[_row_out(), _row_out()],
    )(agg0, degb, hr, Wl0, row(bl0), row(g1), row(be1), Wr1, row(br1))

    agg1 = _sc_agg(h1, src, dst, z128, None, with_deg=False)

    out, emb = pl.pallas_call(
        _tc_last_body,
        grid=(grid,),
        in_specs=[_agg_spec(), _deg_spec(), _row_spec(), _w_spec(), _v_spec(),
                  _v_spec(), _v_spec(), _w_spec(), _v_spec()],
        out_specs=[_row_spec(), _row_spec()],
        out_shape=[_row_out(), _row_out()],
    )(agg1, degb, hr1, Wl1, row(bl1), row(g2), row(be2), W2, row(b2))

    return (out, emb)


# speedup vs baseline: 13.0741x; 1.0828x over previous
import functools

import jax
import jax.numpy as jnp
from jax import lax
from jax.experimental import pallas as pl
from jax.experimental.pallas import tpu as pltpu
from jax.experimental.pallas import tpu_sc as plsc

N = 10000
E = 320000
D = 128
H = 128
OUT = 128
BN_EPS = 1e-5

NC = 2
NS = 16
NPAD = 10240
ROWS_PER_TILE = NPAD // NS
CHUNK = 40
EDGES_PER_CORE = E // NC
EDGES_PER_TILE = E // (NC * NS)
NCHUNK = EDGES_PER_TILE // CHUNK

_mesh = plsc.VectorSubcoreMesh(core_axis_name="c", subcore_axis_name="s")


NSLOT = 5
_MAIN_ITERS = (NCHUNK - 5) // NSLOT
assert 2 + NSLOT * _MAIN_ITERS == NCHUNK - 3


def _sc_agg_body(with_deg, *refs):
    if with_deg:
        (h_hbm, src_hbm, dst_hbm, z128_hbm, z1_hbm,
         agg_out, deg_out,
         agg_sh, deg_sh, src_big, dst_big, ones_v, *rest) = refs
    else:
        (h_hbm, src_hbm, dst_hbm, z128_hbm,
         agg_out,
         agg_sh, src_big, dst_big, *rest) = refs
    dstb = rest[0:NSLOT]
    rows = rest[NSLOT:2 * NSLOT]
    gs = rest[2 * NSLOT:3 * NSLOT]
    ss = rest[3 * NSLOT:4 * NSLOT]

    cid = lax.axis_index("c")
    sid = lax.axis_index("s")
    r0 = sid * ROWS_PER_TILE

    pltpu.sync_copy(z128_hbm.at[pl.ds(r0, ROWS_PER_TILE)],
                    agg_sh.at[pl.ds(r0, ROWS_PER_TILE)])
    if with_deg:
        pltpu.sync_copy(z1_hbm.at[pl.ds(r0, ROWS_PER_TILE)],
                        deg_sh.at[pl.ds(r0, ROWS_PER_TILE)])

        for o in (0, 16, CHUNK - 16):
            ones_v[pl.ds(o, 16)] = jnp.ones((16,), jnp.float32)

    base = cid * EDGES_PER_CORE + sid * EDGES_PER_TILE
    pltpu.sync_copy(src_hbm.at[pl.ds(base, EDGES_PER_TILE)], src_big)
    pltpu.sync_copy(dst_hbm.at[pl.ds(base, EDGES_PER_TILE)], dst_big)

    plsc.subcore_barrier()

    def prep_dst(s, c):
        for o in (0, 16, CHUNK - 16):
            dstb[s][pl.ds(o, 16)] = dst_big[pl.ds(c * CHUNK + o, 16)]

    def g_start(c, s):
        pltpu.async_copy(h_hbm.at[src_big.at[pl.ds(c * CHUNK, CHUNK)]],
                         rows[s], gs[s])

    def g_wait(c, s):
        pltpu.make_async_copy(h_hbm.at[src_big.at[pl.ds(c * CHUNK, CHUNK)]],
                              rows[s], gs[s]).wait()

    def s_start(s):
        pltpu.async_copy(rows[s], agg_sh.at[dstb[s]], ss[s], add=True)
        if with_deg:
            pltpu.sync_copy(ones_v, deg_sh.at[dstb[s]], add=True)

    def s_wait(s):
        pltpu.make_async_copy(rows[s], agg_sh.at[dstb[s]], ss[s]).wait()

    lead = NSLOT - 2
    for c in range(lead):
        prep_dst(c, c)
        g_start(c, c)
    for c in (0, 1):
        g_wait(c, c)
        s_start(c)
        prep_dst((c + lead) % NSLOT, c + lead)
        g_start(c + lead, (c + lead) % NSLOT)

    @pl.loop(0, _MAIN_ITERS)
    def _(k):
        c0 = 2 + NSLOT * k
        for j in range(NSLOT):
            c = c0 + j
            s_cur = (2 + j) % NSLOT
            s_new = j
            g_wait(c, s_cur)
            s_start(s_cur)
            s_wait(s_new)
            prep_dst(s_new, c + lead)
            g_start(c + lead, s_new)

    for c, s_cur, s_old in ((NCHUNK - 3, 2, 0), (NCHUNK - 2, 3, 1),
                            (NCHUNK - 1, 4, 2)):
        g_wait(c, s_cur)
        s_start(s_cur)
        s_wait(s_old)
    s_wait(3)
    s_wait(4)

    plsc.subcore_barrier()

    pltpu.sync_copy(agg_sh.at[pl.ds(r0, ROWS_PER_TILE)],
                    agg_out.at[cid, pl.ds(r0, ROWS_PER_TILE)])
    if with_deg:
        pltpu.sync_copy(deg_sh.at[pl.ds(r0, ROWS_PER_TILE)],
                        deg_out.at[cid, pl.ds(r0, ROWS_PER_TILE)])


def _sc_agg(h, src, dst, z128, z1, with_deg):
    out_type = [jax.ShapeDtypeStruct((NC, NPAD, D), jnp.float32)]
    scratch = [
        pltpu.VMEM_SHARED((NPAD, D), jnp.float32),
    ]
    if with_deg:
        out_type.append(jax.ShapeDtypeStruct((NC, NPAD), jnp.float32))
        scratch.append(pltpu.VMEM_SHARED((NPAD,), jnp.float32))
    scratch += [
        pltpu.VMEM((EDGES_PER_TILE,), jnp.int32),
        pltpu.VMEM((EDGES_PER_TILE,), jnp.int32),
    ]
    if with_deg:
        scratch.append(pltpu.VMEM((CHUNK,), jnp.float32))
    scratch += [pltpu.VMEM((CHUNK,), jnp.int32)] * NSLOT
    scratch += [pltpu.VMEM((CHUNK, D), jnp.float32)] * NSLOT
    scratch += [pltpu.SemaphoreType.DMA] * (2 * NSLOT)

    kern = pl.kernel(
        functools.partial(_sc_agg_body, with_deg),
        out_type=tuple(out_type) if len(out_type) > 1 else out_type[0],
        mesh=_mesh,
        scratch_types=scratch,
    )
    if with_deg:
        return kern(h, src, dst, z128, z1)
    return kern(h, src, dst, z128)


def _bn_scale(g):
    return g * (1.0 / jnp.sqrt(1.0 + BN_EPS))


BLK = 2000


def _tc1_body(x_ref, w1_ref, b1_ref, g0_ref, be0_ref, wr0_ref, br0_ref,
              h_ref, hr_ref):
    h = jnp.dot(x_ref[...], w1_ref[...], preferred_element_type=jnp.float32)
    h = (h + b1_ref[...]) * _bn_scale(g0_ref[...]) + be0_ref[...]
    h_ref[...] = h
    hr_ref[...] = jnp.dot(h, wr0_ref[...],
                          preferred_element_type=jnp.float32) + br0_ref[...]


def _tc_mid_body(agg_ref, deg_ref, hr_ref, wl_ref, bl_ref, g_ref, be_ref,
                 wr_ref, br_ref, h_ref, hrn_ref):
    mean = (agg_ref[0] + agg_ref[1]) / jnp.maximum(deg_ref[...], 1.0)
    t = jnp.dot(mean, wl_ref[...], preferred_element_type=jnp.float32)
    t = t + bl_ref[...] + hr_ref[...]
    h = t * _bn_scale(g_ref[...]) + be_ref[...]
    h_ref[...] = h
    hrn_ref[...] = jnp.dot(h, wr_ref[...],
                           preferred_element_type=jnp.float32) + br_ref[...]


def _tc_last_body(agg_ref, deg_ref, hr_ref, wl_ref, bl_ref, g_ref, be_ref,
                  w2_ref, b2_ref, out_ref, emb_ref):
    mean = (agg_ref[0] + agg_ref[1]) / jnp.maximum(deg_ref[...], 1.0)
    t = jnp.dot(mean, wl_ref[...], preferred_element_type=jnp.float32)
    t = t + bl_ref[...] + hr_ref[...]
    h = t * _bn_scale(g_ref[...]) + be_ref[...]
    emb = jnp.dot(h, w2_ref[...], preferred_element_type=jnp.float32)
    emb = emb + b2_ref[...]
    emb_ref[...] = emb
    m = jnp.max(emb, axis=1, keepdims=True)
    lse = jnp.log(jnp.sum(jnp.exp(emb - m), axis=1, keepdims=True))
    out_ref[...] = emb - m - lse


def _row_spec():
    return pl.BlockSpec((BLK, 128), lambda i: (i, 0))


def _w_spec():
    return pl.BlockSpec((128, 128), lambda i: (0, 0))


def _v_spec():
    return pl.BlockSpec((1, 128), lambda i: (0, 0))


def _agg_spec():
    return pl.BlockSpec((NC, BLK, 128), lambda i: (0, i, 0))


def _deg_spec():
    return _row_spec()


def _row_out(dtype=jnp.float32):
    return jax.ShapeDtypeStruct((N, 128), dtype)


def kernel(x, edge_index, W1, b1, Wl0, bl0, Wr0, br0, Wl1, bl1, Wr1, br1,
           g0, be0, g1, be1, g2, be2, W2, b2):
    src = edge_index[0]
    dst = edge_index[1]
    z128 = jnp.zeros((NPAD, D), jnp.float32)
    z1 = jnp.zeros((NPAD,), jnp.float32)

    row = lambda v: v.reshape(1, 128)
    grid = N // BLK

    h, hr = pl.pallas_call(
        _tc1_body,
        grid=(grid,),
        in_specs=[_row_spec(), _w_spec(), _v_spec(), _v_spec(), _v_spec(),
                  _w_spec(), _v_spec()],
        out_specs=[_row_spec(), _row_spec()],
        out_shape=[_row_out(), _row_out()],
    )(x, W1, row(b1), row(g0), row(be0), Wr0, row(br0))

    agg0, deg = _sc_agg(h, src, dst, z128, z1, with_deg=True)
    degb = jnp.broadcast_to((deg[0, :N] + deg[1, :N])[:, None], (N, 128))

    h1, hr1 = pl.pallas_call(
        _tc_mid_body,
        grid=(grid,),
        in_specs=[_agg_spec(), _deg_spec(), _row_spec(), _w_spec(), _v_spec(),
                  _v_spec(), _v_spec(), _w_spec(), _v_spec()],
        out_specs=[_row_spec(), _row_spec()],
        out_shape=[_row_out(), _row_out()],
    )(agg0, degb, hr, Wl0, row(bl0), row(g1), row(be1), Wr1, row(br1))

    agg1 = _sc_agg(h1, src, dst, z128, None, with_deg=False)

    out, emb = pl.pallas_call(
        _tc_last_body,
        grid=(grid,),
        in_specs=[_agg_spec(), _deg_spec(), _row_spec(), _w_spec(), _v_spec(),
                  _v_spec(), _v_spec(), _w_spec(), _v_spec()],
        out_specs=[_row_spec(), _row_spec()],
        out_shape=[_row_out(), _row_out()],
    )(agg1, degb, hr1, Wl1, row(bl1), row(g2), row(be2), W2, row(b2))

    return (out, emb)


# speedup vs baseline: 13.3969x; 1.0247x over previous
import functools

import jax
import jax.numpy as jnp
from jax import lax
from jax.experimental import pallas as pl
from jax.experimental.pallas import tpu as pltpu
from jax.experimental.pallas import tpu_sc as plsc

N = 10000
E = 320000
D = 128
H = 128
OUT = 128
BN_EPS = 1e-5

NC = 2
NS = 16
NPAD = 10240
ROWS_PER_TILE = NPAD // NS
CHUNK = 40
EDGES_PER_CORE = E // NC
EDGES_PER_TILE = E // (NC * NS)
NCHUNK = EDGES_PER_TILE // CHUNK

_mesh = plsc.VectorSubcoreMesh(core_axis_name="c", subcore_axis_name="s")


NSLOT = 5
_MAIN_ITERS = (NCHUNK - 5) // NSLOT
assert 2 + NSLOT * _MAIN_ITERS == NCHUNK - 3


def _sc_agg_body(with_deg, *refs):
    if with_deg:
        (h_hbm, src_hbm, dst_hbm, z128_hbm, z1_hbm,
         agg_out, deg_out,
         agg_sh, deg_sh, src_big, dst_big, ones_v, *rest) = refs
    else:
        (h_hbm, src_hbm, dst_hbm, z128_hbm,
         agg_out,
         agg_sh, src_big, dst_big, *rest) = refs
    dstb = rest[0:NSLOT]
    rows = rest[NSLOT:2 * NSLOT]
    gs = rest[2 * NSLOT:3 * NSLOT]
    ss = rest[3 * NSLOT:4 * NSLOT]

    cid = lax.axis_index("c")
    sid = lax.axis_index("s")
    r0 = sid * ROWS_PER_TILE

    pltpu.sync_copy(z128_hbm.at[pl.ds(r0, ROWS_PER_TILE)],
                    agg_sh.at[pl.ds(r0, ROWS_PER_TILE)])
    if with_deg:
        pltpu.sync_copy(z1_hbm.at[pl.ds(r0, ROWS_PER_TILE)],
                        deg_sh.at[pl.ds(r0, ROWS_PER_TILE)])

        for o in (0, 16, CHUNK - 16):
            ones_v[pl.ds(o, 16)] = jnp.ones((16,), jnp.float32)

    base = cid * EDGES_PER_CORE + sid * EDGES_PER_TILE
    pltpu.sync_copy(src_hbm.at[pl.ds(base, EDGES_PER_TILE)], src_big)
    pltpu.sync_copy(dst_hbm.at[pl.ds(base, EDGES_PER_TILE)], dst_big)

    plsc.subcore_barrier()

    def prep_dst(s, c):
        for o in (0, 16, CHUNK - 16):
            dstb[s][pl.ds(o, 16)] = dst_big[pl.ds(c * CHUNK + o, 16)]

    def g_start(c, s):
        pltpu.async_copy(h_hbm.at[src_big.at[pl.ds(c * CHUNK, CHUNK)]],
                         rows[s], gs[s])

    def g_wait(c, s):
        pltpu.make_async_copy(h_hbm.at[src_big.at[pl.ds(c * CHUNK, CHUNK)]],
                              rows[s], gs[s]).wait()

    def s_start(s):
        pltpu.async_copy(rows[s], agg_sh.at[dstb[s]], ss[s], add=True)
        if with_deg:
            pltpu.sync_copy(ones_v, deg_sh.at[dstb[s]], add=True)

    def s_wait(s):
        pltpu.make_async_copy(rows[s], agg_sh.at[dstb[s]], ss[s]).wait()

    lead = NSLOT - 2
    for c in range(lead):
        prep_dst(c, c)
        g_start(c, c)
    for c in (0, 1):
        g_wait(c, c)
        s_start(c)
        prep_dst((c + lead) % NSLOT, c + lead)
        g_start(c + lead, (c + lead) % NSLOT)

    @pl.loop(0, _MAIN_ITERS)
    def _(k):
        c0 = 2 + NSLOT * k
        for j in range(NSLOT):
            c = c0 + j
            s_cur = (2 + j) % NSLOT
            s_new = j
            g_wait(c, s_cur)
            s_start(s_cur)
            s_wait(s_new)
            prep_dst(s_new, c + lead)
            g_start(c + lead, s_new)

    for c, s_cur, s_old in ((NCHUNK - 3, 2, 0), (NCHUNK - 2, 3, 1),
                            (NCHUNK - 1, 4, 2)):
        g_wait(c, s_cur)
        s_start(s_cur)
        s_wait(s_old)
    s_wait(3)
    s_wait(4)

    plsc.subcore_barrier()

    pltpu.sync_copy(agg_sh.at[pl.ds(r0, ROWS_PER_TILE)],
                    agg_out.at[cid, pl.ds(r0, ROWS_PER_TILE)])
    if with_deg:
        pltpu.sync_copy(deg_sh.at[pl.ds(r0, ROWS_PER_TILE)],
                        deg_out.at[cid, pl.ds(r0, ROWS_PER_TILE)])


def _sc_agg(h, src, dst, z128, z1, with_deg):
    out_type = [jax.ShapeDtypeStruct((NC, NPAD, D), jnp.float32)]
    scratch = [
        pltpu.VMEM_SHARED((NPAD, D), jnp.float32),
    ]
    if with_deg:
        out_type.append(jax.ShapeDtypeStruct((NC, NPAD), jnp.float32))
        scratch.append(pltpu.VMEM_SHARED((NPAD,), jnp.float32))
    scratch += [
        pltpu.VMEM((EDGES_PER_TILE,), jnp.int32),
        pltpu.VMEM((EDGES_PER_TILE,), jnp.int32),
    ]
    if with_deg:
        scratch.append(pltpu.VMEM((CHUNK,), jnp.float32))
    scratch += [pltpu.VMEM((CHUNK,), jnp.int32)] * NSLOT
    scratch += [pltpu.VMEM((CHUNK, D), jnp.float32)] * NSLOT
    scratch += [pltpu.SemaphoreType.DMA] * (2 * NSLOT)

    kern = pl.kernel(
        functools.partial(_sc_agg_body, with_deg),
        out_type=tuple(out_type) if len(out_type) > 1 else out_type[0],
        mesh=_mesh,
        scratch_types=scratch,
    )
    if with_deg:
        return kern(h, src, dst, z128, z1)
    return kern(h, src, dst, z128)


def _bn_scale(g):
    return g * (1.0 / jnp.sqrt(1.0 + BN_EPS))


BLK = 2048


def _tc1_body(x_ref, w1_ref, b1_ref, g0_ref, be0_ref, wr0_ref, br0_ref,
              h_ref, hr_ref):
    h = jnp.dot(x_ref[...], w1_ref[...], preferred_element_type=jnp.float32)
    h = (h + b1_ref[...]) * _bn_scale(g0_ref[...]) + be0_ref[...]
    h_ref[...] = h
    hr_ref[...] = jnp.dot(h, wr0_ref[...],
                          preferred_element_type=jnp.float32) + br0_ref[...]


def _tc_mid_body(agg_ref, deg_ref, hr_ref, wl_ref, bl_ref, g_ref, be_ref,
                 wr_ref, br_ref, h_ref, hrn_ref):
    d = jnp.maximum(deg_ref[0, :] + deg_ref[1, :], 1.0)
    mean = (agg_ref[0] + agg_ref[1]) / d[:, None]
    t = jnp.dot(mean, wl_ref[...], preferred_element_type=jnp.float32)
    t = t + bl_ref[...] + hr_ref[...]
    h = t * _bn_scale(g_ref[...]) + be_ref[...]
    h_ref[...] = h
    hrn_ref[...] = jnp.dot(h, wr_ref[...],
                           preferred_element_type=jnp.float32) + br_ref[...]


def _tc_last_body(agg_ref, deg_ref, hr_ref, wl_ref, bl_ref, g_ref, be_ref,
                  w2_ref, b2_ref, out_ref, emb_ref):
    d = jnp.maximum(deg_ref[0, :] + deg_ref[1, :], 1.0)
    mean = (agg_ref[0] + agg_ref[1]) / d[:, None]
    t = jnp.dot(mean, wl_ref[...], preferred_element_type=jnp.float32)
    t = t + bl_ref[...] + hr_ref[...]
    h = t * _bn_scale(g_ref[...]) + be_ref[...]
    emb = jnp.dot(h, w2_ref[...], preferred_element_type=jnp.float32)
    emb = emb + b2_ref[...]
    emb_ref[...] = emb
    m = jnp.max(emb, axis=1, keepdims=True)
    lse = jnp.log(jnp.sum(jnp.exp(emb - m), axis=1, keepdims=True))
    out_ref[...] = emb - m - lse


def _row_spec():
    return pl.BlockSpec((BLK, 128), lambda i: (i, 0))


def _w_spec():
    return pl.BlockSpec((128, 128), lambda i: (0, 0))


def _v_spec():
    return pl.BlockSpec((1, 128), lambda i: (0, 0))


def _agg_spec():
    return pl.BlockSpec((NC, BLK, 128), lambda i: (0, i, 0))


def _deg_spec():
    return pl.BlockSpec((NC, BLK), lambda i: (0, i))


def _row_out(dtype=jnp.float32):
    return jax.ShapeDtypeStruct((N, 128), dtype)


def kernel(x, edge_index, W1, b1, Wl0, bl0, Wr0, br0, Wl1, bl1, Wr1, br1,
           g0, be0, g1, be1, g2, be2, W2, b2):
    src = edge_index[0]
    dst = edge_index[1]
    z128 = jnp.zeros((NPAD, D), jnp.float32)
    z1 = jnp.zeros((NPAD,), jnp.float32)

    row = lambda v: v.reshape(1, 128)
    grid = (N + BLK - 1) // BLK

    h, hr = pl.pallas_call(
        _tc1_body,
        grid=(grid,),
        in_specs=[_row_spec(), _w_spec(), _v_spec(), _v_spec(), _v_spec(),
                  _w_spec(), _v_spec()],
        out_specs=[_row_spec(), _row_spec()],
        out_shape=[_row_out(), _row_out()],
    )(x, W1, row(b1), row(g0), row(be0), Wr0, row(br0))

    agg0, deg = _sc_agg(h, src, dst, z128, z1, with_deg=True)

    h1, hr1 = pl.pallas_call(
        _tc_mid_body,
        grid=(grid,),
        in_specs=[_agg_spec(), _deg_spec(), _row_spec(), _w_spec(), _v_spec(),
                  _v_spec(), _v_spec(), _w_spec(), _v_spec()],
        out_specs=[_row_spec(), _row_spec()],
        out_shape=[_row_out(), _row_out()],
    )(agg0, deg, hr, Wl0, row(bl0), row(g1), row(be1), Wr1, row(br1))

    agg1 = _sc_agg(h1, src, dst, z128, None, with_deg=False)

    out, emb = pl.pallas_call(
        _tc_last_body,
        grid=(grid,),
        in_specs=[_agg_spec(), _deg_spec(), _row_spec(), _w_spec(), _v_spec(),
                  _v_spec(), _v_spec(), _w_spec(), _v_spec()],
        out_specs=[_row_spec(), _row_spec()],
        out_shape=[_row_out(), _row_out()],
    )(agg1, deg, hr1, Wl1, row(bl1), row(g2), row(be2), W2, row(b2))

    return (out, emb)


# speedup vs baseline: 13.6631x; 1.0199x over previous
import functools

import jax
import jax.numpy as jnp
from jax import lax
from jax.experimental import pallas as pl
from jax.experimental.pallas import tpu as pltpu
from jax.experimental.pallas import tpu_sc as plsc

N = 10000
E = 320000
D = 128
H = 128
OUT = 128
BN_EPS = 1e-5

NC = 2
NS = 16
NPAD = 10240
ROWS_PER_TILE = NPAD // NS
CHUNK = 40
EDGES_PER_CORE = E // NC
EDGES_PER_TILE = E // (NC * NS)
NCHUNK = EDGES_PER_TILE // CHUNK

_mesh = plsc.VectorSubcoreMesh(core_axis_name="c", subcore_axis_name="s")


NSLOT = 5
_MAIN_ITERS = (NCHUNK - 5) // NSLOT
assert 2 + NSLOT * _MAIN_ITERS == NCHUNK - 3


def _sc_agg_body(with_deg, *refs):
    if with_deg:
        (h_hbm, src_hbm, dst_hbm, z128_hbm, z1_hbm,
         agg_out, deg_out,
         agg_sh, deg_sh, src_big, dst_big, ones_v, *rest) = refs
    else:
        (h_hbm, src_hbm, dst_hbm, z128_hbm,
         agg_out,
         agg_sh, src_big, dst_big, *rest) = refs
    dstb = rest[0:NSLOT]
    rows = rest[NSLOT:2 * NSLOT]
    gs = rest[2 * NSLOT:3 * NSLOT]
    ss = rest[3 * NSLOT:4 * NSLOT]

    cid = lax.axis_index("c")
    sid = lax.axis_index("s")
    r0 = sid * ROWS_PER_TILE

    base = cid * EDGES_PER_CORE + sid * EDGES_PER_TILE
    pend = [
        pltpu.async_copy(z128_hbm.at[pl.ds(r0, ROWS_PER_TILE)],
                         agg_sh.at[pl.ds(r0, ROWS_PER_TILE)], gs[0]),
        pltpu.async_copy(src_hbm.at[pl.ds(base, EDGES_PER_TILE)],
                         src_big, gs[1]),
        pltpu.async_copy(dst_hbm.at[pl.ds(base, EDGES_PER_TILE)],
                         dst_big, gs[2]),
    ]
    if with_deg:
        pend.append(pltpu.async_copy(z1_hbm.at[pl.ds(r0, ROWS_PER_TILE)],
                                     deg_sh.at[pl.ds(r0, ROWS_PER_TILE)],
                                     ss[0]))
        for o in (0, 16, CHUNK - 16):
            ones_v[pl.ds(o, 16)] = jnp.ones((16,), jnp.float32)
    for p in pend:
        p.wait()

    plsc.subcore_barrier()

    def prep_dst(s, c):
        for o in (0, 16, CHUNK - 16):
            dstb[s][pl.ds(o, 16)] = dst_big[pl.ds(c * CHUNK + o, 16)]

    def g_start(c, s):
        pltpu.async_copy(h_hbm.at[src_big.at[pl.ds(c * CHUNK, CHUNK)]],
                         rows[s], gs[s])

    def g_wait(c, s):
        pltpu.make_async_copy(h_hbm.at[src_big.at[pl.ds(c * CHUNK, CHUNK)]],
                              rows[s], gs[s]).wait()

    def s_start(s):
        pltpu.async_copy(rows[s], agg_sh.at[dstb[s]], ss[s], add=True)
        if with_deg:
            pltpu.sync_copy(ones_v, deg_sh.at[dstb[s]], add=True)

    def s_wait(s):
        pltpu.make_async_copy(rows[s], agg_sh.at[dstb[s]], ss[s]).wait()

    lead = NSLOT - 2
    for c in range(lead):
        prep_dst(c, c)
        g_start(c, c)
    for c in (0, 1):
        g_wait(c, c)
        s_start(c)
        prep_dst((c + lead) % NSLOT, c + lead)
        g_start(c + lead, (c + lead) % NSLOT)

    @pl.loop(0, _MAIN_ITERS)
    def _(k):
        c0 = 2 + NSLOT * k
        for j in range(NSLOT):
            c = c0 + j
            s_cur = (2 + j) % NSLOT
            s_new = j
            g_wait(c, s_cur)
            s_start(s_cur)
            s_wait(s_new)
            prep_dst(s_new, c + lead)
            g_start(c + lead, s_new)

    for c, s_cur, s_old in ((NCHUNK - 3, 2, 0), (NCHUNK - 2, 3, 1),
                            (NCHUNK - 1, 4, 2)):
        g_wait(c, s_cur)
        s_start(s_cur)
        s_wait(s_old)
    s_wait(3)
    s_wait(4)

    plsc.subcore_barrier()

    pend = [pltpu.async_copy(agg_sh.at[pl.ds(r0, ROWS_PER_TILE)],
                             agg_out.at[cid, pl.ds(r0, ROWS_PER_TILE)], gs[0])]
    if with_deg:
        pend.append(pltpu.async_copy(deg_sh.at[pl.ds(r0, ROWS_PER_TILE)],
                                     deg_out.at[cid, pl.ds(r0, ROWS_PER_TILE)],
                                     gs[1]))
    for p in pend:
        p.wait()


def _sc_agg(h, src, dst, z128, z1, with_deg):
    out_type = [jax.ShapeDtypeStruct((NC, NPAD, D), jnp.float32)]
    scratch = [
        pltpu.VMEM_SHARED((NPAD, D), jnp.float32),
    ]
    if with_deg:
        out_type.append(jax.ShapeDtypeStruct((NC, NPAD), jnp.float32))
        scratch.append(pltpu.VMEM_SHARED((NPAD,), jnp.float32))
    scratch += [
        pltpu.VMEM((EDGES_PER_TILE,), jnp.int32),
        pltpu.VMEM((EDGES_PER_TILE,), jnp.int32),
    ]
    if with_deg:
        scratch.append(pltpu.VMEM((CHUNK,), jnp.float32))
    scratch += [pltpu.VMEM((CHUNK,), jnp.int32)] * NSLOT
    scratch += [pltpu.VMEM((CHUNK, D), jnp.float32)] * NSLOT
    scratch += [pltpu.SemaphoreType.DMA] * (2 * NSLOT)

    kern = pl.kernel(
        functools.partial(_sc_agg_body, with_deg),
        out_type=tuple(out_type) if len(out_type) > 1 else out_type[0],
        mesh=_mesh,
        scratch_types=scratch,
    )
    if with_deg:
        return kern(h, src, dst, z128, z1)
    return kern(h, src, dst, z128)


def _bn_scale(g):
    return g * (1.0 / jnp.sqrt(1.0 + BN_EPS))


BLK = 2048


def _tc1_body(x_ref, w1_ref, b1_ref, g0_ref, be0_ref, wr0_ref, br0_ref,
              h_ref, hr_ref):
    h = jnp.dot(x_ref[...], w1_ref[...], preferred_element_type=jnp.float32)
    h = (h + b1_ref[...]) * _bn_scale(g0_ref[...]) + be0_ref[...]
    h_ref[...] = h
    hr_ref[...] = jnp.dot(h, wr0_ref[...],
                          preferred_element_type=jnp.float32) + br0_ref[...]


def _tc_mid_body(agg_ref, deg_ref, hr_ref, wl_ref, bl_ref, g_ref, be_ref,
                 wr_ref, br_ref, h_ref, hrn_ref):
    d = jnp.maximum(deg_ref[0, :] + deg_ref[1, :], 1.0)
    mean = (agg_ref[0] + agg_ref[1]) / d[:, None]
    t = jnp.dot(mean, wl_ref[...], preferred_element_type=jnp.float32)
    t = t + bl_ref[...] + hr_ref[...]
    h = t * _bn_scale(g_ref[...]) + be_ref[...]
    h_ref[...] = h
    hrn_ref[...] = jnp.dot(h, wr_ref[...],
                           preferred_element_type=jnp.float32) + br_ref[...]


def _tc_last_body(agg_ref, deg_ref, hr_ref, wl_ref, bl_ref, g_ref, be_ref,
                  w2_ref, b2_ref, out_ref, emb_ref):
    d = jnp.maximum(deg_ref[0, :] + deg_ref[1, :], 1.0)
    mean = (agg_ref[0] + agg_ref[1]) / d[:, None]
    t = jnp.dot(mean, wl_ref[...], preferred_element_type=jnp.float32)
    t = t + bl_ref[...] + hr_ref[...]
    h = t * _bn_scale(g_ref[...]) + be_ref[...]
    emb = jnp.dot(h, w2_ref[...], preferred_element_type=jnp.float32)
    emb = emb + b2_ref[...]
    emb_ref[...] = emb
    m = jnp.max(emb, axis=1, keepdims=True)
    lse = jnp.log(jnp.sum(jnp.exp(emb - m), axis=1, keepdims=True))
    out_ref[...] = emb - m - lse


def _row_spec():
    return pl.BlockSpec((BLK, 128), lambda i: (i, 0))


def _w_spec():
    return pl.BlockSpec((128, 128), lambda i: (0, 0))


def _v_spec():
    return pl.BlockSpec((1, 128), lambda i: (0, 0))


def _agg_spec():
    return pl.BlockSpec((NC, BLK, 128), lambda i: (0, i, 0))


def _deg_spec():
    return pl.BlockSpec((NC, BLK), lambda i: (0, i))


def _row_out(dtype=jnp.float32):
    return jax.ShapeDtypeStruct((N, 128), dtype)


def kernel(x, edge_index, W1, b1, Wl0, bl0, Wr0, br0, Wl1, bl1, Wr1, br1,
           g0, be0, g1, be1, g2, be2, W2, b2):
    src = edge_index[0]
    dst = edge_index[1]
    z128 = jnp.zeros((NPAD, D), jnp.float32)
    z1 = jnp.zeros((NPAD,), jnp.float32)

    row = lambda v: v.reshape(1, 128)
    grid = (N + BLK - 1) // BLK

    h, hr = pl.pallas_call(
        _tc1_body,
        grid=(grid,),
        in_specs=[_row_spec(), _w_spec(), _v_spec(), _v_spec(), _v_spec(),
                  _w_spec(), _v_spec()],
        out_specs=[_row_spec(), _row_spec()],
        out_shape=[_row_out(), _row_out()],
    )(x, W1, row(b1), row(g0), row(be0), Wr0, row(br0))

    agg0, deg = _sc_agg(h, src, dst, z128, z1, with_deg=True)

    h1, hr1 = pl.pallas_call(
        _tc_mid_body,
        grid=(grid,),
        in_specs=[_agg_spec(), _deg_spec(), _row_spec(), _w_spec(), _v_spec(),
                  _v_spec(), _v_spec(), _w_spec(), _v_spec()],
        out_specs=[_row_spec(), _row_spec()],
        out_shape=[_row_out(), _row_out()],
    )(agg0, deg, hr, Wl0, row(bl0), row(g1), row(be1), Wr1, row(br1))

    agg1 = _sc_agg(h1, src, dst, z128, None, with_deg=False)

    out, emb = pl.pallas_call(
        _tc_last_body,
        grid=(grid,),
        in_specs=[_agg_spec(), _deg_spec(), _row_spec(), _w_spec(), _v_spec(),
                  _v_spec(), _v_spec(), _w_spec(), _v_spec()],
        out_specs=[_row_spec(), _row_spec()],
        out_shape=[_row_out(), _row_out()],
    )(agg1, deg, hr1, Wl1, row(bl1), row(g2), row(be2), W2, row(b2))

    return (out, emb)
